# Initial kernel scaffold; baseline (speedup 1.0000x reference)
import jax
import jax.numpy as jnp
from jax.experimental import pallas as pl


def kernel(positions, cell):
    raise NotImplementedError("write your pallas kernel here")


# speedup vs baseline: 6.4604x; 6.4604x over previous
import functools

import jax
import jax.numpy as jnp
from jax import lax
from jax.experimental import pallas as pl
from jax.experimental.pallas import tpu as pltpu
from jax.experimental.pallas import tpu_sc as plsc

N = 2048
K = 19
NC = 2
NS = 16
L = 16
NW = NC * NS
ROWS_PER_W = N // NW
NGROUPS = ROWS_PER_W // L
NSEG = N // L


def _splat_f(v):
    return jnp.full((L,), v, jnp.float32)


def _splat_i(v):
    return jnp.full((L,), v, jnp.int32)


def _knn_sc(px, py, pz, cellv):
    mesh = plsc.VectorSubcoreMesh(
        core_axis_name="c", subcore_axis_name="s", num_cores=NC, num_subcores=NS
    )

    @functools.partial(
        pl.kernel,
        out_type=(
            jax.ShapeDtypeStruct((N, K), jnp.int32),
            jax.ShapeDtypeStruct((N, K), jnp.float32),
            jax.ShapeDtypeStruct((N, 3 * K), jnp.float32),
        ),
        mesh=mesh,
        compiler_params=pltpu.CompilerParams(needs_layout_passes=False),
        scratch_types=[
            pltpu.VMEM((N,), jnp.float32),
            pltpu.VMEM((N,), jnp.float32),
            pltpu.VMEM((N,), jnp.float32),
            pltpu.VMEM((L,), jnp.float32),
            pltpu.VMEM((N * L,), jnp.float32),
            pltpu.VMEM((NSEG * L,), jnp.float32),
            pltpu.VMEM((L, K), jnp.int32),
            pltpu.VMEM((L, K), jnp.float32),
            pltpu.VMEM((L, 3 * K), jnp.float32),
        ],
    )
    def knn_kernel(px_h, py_h, pz_h, cell_h, oidx_h, odist_h, ovec_h,
                   px_v, py_v, pz_v, cell_v, buf, segmin, idx_st, dist_st, vec_st):
        wid = lax.axis_index("s") * NC + lax.axis_index("c")
        pltpu.sync_copy(px_h, px_v)
        pltpu.sync_copy(py_h, py_v)
        pltpu.sync_copy(pz_h, pz_v)
        pltpu.sync_copy(cell_h, cell_v)

        lanes = lax.iota(jnp.int32, L)
        half = _splat_f(0.5)
        one = _splat_f(1.0)
        zero = _splat_f(0.0)
        inf = _splat_f(jnp.inf)

        def bget(e):
            return plsc.load_gather(cell_v, [_splat_i(e + 1)])

        c00, c01, c02 = bget(0), bget(1), bget(2)
        c10, c11, c12 = bget(3), bget(4), bget(5)
        c20, c21, c22 = bget(6), bget(7), bget(8)

        def wrap(d):
            adj = jnp.where(jnp.abs(d) > half, jnp.sign(d), zero)
            return d - adj

        def bf16r(x):
            bits = plsc.bitcast(x, jnp.int32)
            r = bits + (jnp.int32(0x7FFF) + ((bits >> 16) & 1))
            return plsc.bitcast(r & jnp.int32(-65536), jnp.float32)

        def cart_of(dx, dy, dz):
            dx, dy, dz = bf16r(dx), bf16r(dy), bf16r(dz)
            cx = dx * c00 + dy * c10 + dz * c20
            cy = dx * c01 + dy * c11 + dz * c21
            cz = dx * c02 + dy * c12 + dz * c22
            return cx, cy, cz

        def group_body(g, _):
            row0 = wid * ROWS_PER_W + g * L
            rx = px_v[pl.ds(row0, L)]
            ry = py_v[pl.ds(row0, L)]
            rz = pz_v[pl.ds(row0, L)]

            def seg_body(s, _):
                j0 = s * L
                sm = inf
                for t in range(L):
                    j = j0 + t
                    jsp = jnp.full((L,), j, jnp.int32)
                    pjx = plsc.load_gather(px_v, [jsp])
                    pjy = plsc.load_gather(py_v, [jsp])
                    pjz = plsc.load_gather(pz_v, [jsp])
                    dx = wrap(rx - pjx)
                    dy = wrap(ry - pjy)
                    dz = wrap(rz - pjz)
                    cx, cy, cz = cart_of(dx, dy, dz)
                    d2 = cx * cx + cy * cy + cz * cz
                    buf[pl.ds(j * L, L)] = d2
                    sm = jnp.minimum(sm, d2)
                segmin[pl.ds(s * L, L)] = sm
                return _
            lax.fori_loop(0, NSEG, seg_body, None, unroll=False)

            riv = row0 + lanes
            plsc.store_scatter(buf, [riv * L + lanes], inf)
            sm = inf
            for t in range(L):
                sm = jnp.minimum(sm, buf[pl.ds((row0 + t) * L, L)])
            segmin[pl.ds((row0 // L) * L, L)] = sm

            def rank_body(kk, _):
                def scan_body(s, carry):
                    m, sidx = carry
                    v = segmin[pl.ds(s * L, L)]
                    upd = v < m
                    m = jnp.where(upd, v, m)
                    sidx = jnp.where(upd, jnp.full((L,), s, jnp.int32), sidx)
                    return m, sidx
                m, sidx = lax.fori_loop(
                    0, NSEG, scan_body, (inf, _splat_i(0)), unroll=False)

                sbase = sidx * L
                abase = sbase * L + lanes
                vts = [plsc.load_gather(buf, [abase + (t * L)]) for t in range(L)]
                jstar = sbase + (L - 1)
                for t in range(L - 1, -1, -1):
                    jstar = jnp.where(vts[t] == m, sbase + t, jstar)

                plsc.store_scatter(buf, [jstar * L + lanes], inf)
                sm2 = inf
                for t in range(L):
                    sm2 = jnp.minimum(
                        sm2, jnp.where(sbase + t == jstar, inf, vts[t]))
                plsc.store_scatter(segmin, [sidx * L + lanes], sm2)

                pjx = plsc.load_gather(px_v, [jstar])
                pjy = plsc.load_gather(py_v, [jstar])
                pjz = plsc.load_gather(pz_v, [jstar])
                dx = wrap(rx - pjx)
                dy = wrap(ry - pjy)
                dz = wrap(rz - pjz)
                cx, cy, cz = cart_of(dx, dy, dz)

                d2c = jnp.maximum(m, zero)
                bits = plsc.bitcast(d2c, jnp.int32)
                y = plsc.bitcast(_splat_i(0x5F3759DF) - (bits >> 1), jnp.float32)
                hx = 0.5 * d2c
                y = y * (1.5 - hx * y * y)
                y = y * (1.5 - hx * y * y)
                sr = d2c * y
                sr = 0.5 * (sr + d2c / sr)
                dist = jnp.where(d2c > zero, sr, zero)

                ksp = jnp.full((L,), kk, jnp.int32)
                plsc.store_scatter(idx_st, [lanes, ksp], jstar)
                plsc.store_scatter(dist_st, [lanes, ksp], dist)
                k3 = ksp * 3
                plsc.store_scatter(vec_st, [lanes, k3], cx)
                plsc.store_scatter(vec_st, [lanes, k3 + 1], cy)
                plsc.store_scatter(vec_st, [lanes, k3 + 2], cz)
                return _
            lax.fori_loop(0, K, rank_body, None, unroll=False)

            pltpu.sync_copy(idx_st, oidx_h.at[pl.ds(row0, L)])
            pltpu.sync_copy(dist_st, odist_h.at[pl.ds(row0, L)])
            pltpu.sync_copy(vec_st, ovec_h.at[pl.ds(row0, L)])
            return _

        lax.fori_loop(0, NGROUPS, group_body, None, unroll=False)

    return knn_kernel(px, py, pz, cellv)


def kernel(positions, cell):
    px = positions[:, 0]
    py = positions[:, 1]
    pz = positions[:, 2]
    cellb = cell.astype(jnp.bfloat16).astype(jnp.float32)
    cellv = jnp.zeros((L,), jnp.float32).at[1:10].set(cellb.reshape(9))
    oidx, od, ov = _knn_sc(px, py, pz, cellv)
    src = jnp.broadcast_to(jnp.arange(N, dtype=jnp.int32)[:, None], (N, K))
    edge_index = jnp.stack([src.reshape(-1), oidx.reshape(-1)], axis=0)
    edge_vec = ov.reshape(N * K, 3)
    edge_dist = od.reshape(-1)
    return edge_index, edge_vec, edge_dist


# speedup vs baseline: 7.5142x; 1.1631x over previous
import functools

import jax
import jax.numpy as jnp
from jax import lax
from jax.experimental import pallas as pl
from jax.experimental.pallas import tpu as pltpu
from jax.experimental.pallas import tpu_sc as plsc

N = 2048
K = 19
NC = 2
NS = 16
L = 16
NW = NC * NS
ROWS_PER_W = N // NW
NGROUPS = ROWS_PER_W // L
NSEG = N // L
SEG_PER_BLK = 8
NBLK = NSEG // SEG_PER_BLK


def _splat_f(v):
    return jnp.full((L,), v, jnp.float32)


def _splat_i(v):
    return jnp.full((L,), v, jnp.int32)


def _knn_sc(px, py, pz, cellv):
    mesh = plsc.VectorSubcoreMesh(
        core_axis_name="c", subcore_axis_name="s", num_cores=NC, num_subcores=NS
    )

    @functools.partial(
        pl.kernel,
        out_type=(
            jax.ShapeDtypeStruct((N, K), jnp.int32),
            jax.ShapeDtypeStruct((N, K), jnp.float32),
            jax.ShapeDtypeStruct((N, 3 * K), jnp.float32),
        ),
        mesh=mesh,
        compiler_params=pltpu.CompilerParams(needs_layout_passes=False),
        scratch_types=[
            pltpu.VMEM((N,), jnp.float32),
            pltpu.VMEM((N,), jnp.float32),
            pltpu.VMEM((N,), jnp.float32),
            pltpu.VMEM((L,), jnp.float32),
            pltpu.VMEM((N * L,), jnp.float32),
            pltpu.VMEM((NSEG * L,), jnp.float32),
            pltpu.VMEM((NBLK * L,), jnp.float32),
            pltpu.VMEM((L, K), jnp.int32),
            pltpu.VMEM((L, K), jnp.float32),
            pltpu.VMEM((L, 3 * K), jnp.float32),
        ],
    )
    def knn_kernel(px_h, py_h, pz_h, cell_h, oidx_h, odist_h, ovec_h,
                   px_v, py_v, pz_v, cell_v, buf, segmin, blkmin,
                   idx_st, dist_st, vec_st):
        wid = lax.axis_index("s") * NC + lax.axis_index("c")
        pltpu.sync_copy(px_h, px_v)
        pltpu.sync_copy(py_h, py_v)
        pltpu.sync_copy(pz_h, pz_v)
        pltpu.sync_copy(cell_h, cell_v)

        lanes = lax.iota(jnp.int32, L)
        half = _splat_f(0.5)
        one = _splat_f(1.0)
        zero = _splat_f(0.0)
        inf = _splat_f(jnp.inf)

        def bget(e):
            return plsc.load_gather(cell_v, [_splat_i(e + 1)])

        c00, c01, c02 = bget(0), bget(1), bget(2)
        c10, c11, c12 = bget(3), bget(4), bget(5)
        c20, c21, c22 = bget(6), bget(7), bget(8)

        def wrap(d):
            adj = jnp.where(jnp.abs(d) > half, jnp.sign(d), zero)
            return d - adj

        def bf16r(x):
            bits = plsc.bitcast(x, jnp.int32)
            r = bits + (jnp.int32(0x7FFF) + ((bits >> 16) & 1))
            return plsc.bitcast(r & jnp.int32(-65536), jnp.float32)

        def cart_of(dx, dy, dz):
            dx, dy, dz = bf16r(dx), bf16r(dy), bf16r(dz)
            cx = dx * c00 + dy * c10 + dz * c20
            cy = dx * c01 + dy * c11 + dz * c21
            cz = dx * c02 + dy * c12 + dz * c22
            return cx, cy, cz

        def group_body(g, _):
            row0 = wid * ROWS_PER_W + g * L
            rx = px_v[pl.ds(row0, L)]
            ry = py_v[pl.ds(row0, L)]
            rz = pz_v[pl.ds(row0, L)]

            def seg_body(s, _):
                j0 = s * L
                sm = inf
                for t in range(L):
                    j = j0 + t
                    jsp = jnp.full((L,), j, jnp.int32)
                    pjx = plsc.load_gather(px_v, [jsp])
                    pjy = plsc.load_gather(py_v, [jsp])
                    pjz = plsc.load_gather(pz_v, [jsp])
                    dx = wrap(rx - pjx)
                    dy = wrap(ry - pjy)
                    dz = wrap(rz - pjz)
                    cx, cy, cz = cart_of(dx, dy, dz)
                    d2 = cx * cx + cy * cy + cz * cz
                    buf[pl.ds(j * L, L)] = d2
                    sm = jnp.minimum(sm, d2)
                segmin[pl.ds(s * L, L)] = sm
                return _
            lax.fori_loop(0, NSEG, seg_body, None, unroll=False)

            riv = row0 + lanes
            plsc.store_scatter(buf, [riv * L + lanes], inf)
            sm = inf
            for t in range(L):
                sm = jnp.minimum(sm, buf[pl.ds((row0 + t) * L, L)])
            segmin[pl.ds((row0 // L) * L, L)] = sm

            def blk_body(b, _):
                bm = inf
                for u in range(SEG_PER_BLK):
                    bm = jnp.minimum(bm, segmin[pl.ds((b * SEG_PER_BLK + u) * L, L)])
                blkmin[pl.ds(b * L, L)] = bm
                return _
            lax.fori_loop(0, NBLK, blk_body, None, unroll=False)

            def rank_body(kk, _):
                chains = []
                for c in range(4):
                    m = blkmin[pl.ds((4 * c) * L, L)]
                    bi = _splat_i(4 * c)
                    for b in range(4 * c + 1, 4 * c + 4):
                        v = blkmin[pl.ds(b * L, L)]
                        upd = v < m
                        m = jnp.where(upd, v, m)
                        bi = jnp.where(upd, _splat_i(b), bi)
                    chains.append((m, bi))
                m, bstar = chains[0]
                for cm, cb in chains[1:]:
                    upd = cm < m
                    m = jnp.where(upd, cm, m)
                    bstar = jnp.where(upd, cb, bstar)
                sseg0 = bstar * SEG_PER_BLK
                sv0 = plsc.load_gather(segmin, [sseg0 * L + lanes])
                sidx = sseg0
                for u in range(1, SEG_PER_BLK):
                    v = plsc.load_gather(segmin, [(sseg0 + u) * L + lanes])
                    upd = v < sv0
                    sv0 = jnp.where(upd, v, sv0)
                    sidx = jnp.where(upd, sseg0 + u, sidx)
                m = sv0

                sbase = sidx * L
                abase = sbase * L + lanes
                vts = [plsc.load_gather(buf, [abase + (t * L)]) for t in range(L)]
                jstar = sbase + (L - 1)
                for t in range(L - 1, -1, -1):
                    jstar = jnp.where(vts[t] == m, sbase + t, jstar)

                plsc.store_scatter(buf, [jstar * L + lanes], inf)
                sm2 = inf
                for t in range(L):
                    sm2 = jnp.minimum(
                        sm2, jnp.where(sbase + t == jstar, inf, vts[t]))
                plsc.store_scatter(segmin, [sidx * L + lanes], sm2)
                bm2 = inf
                for u in range(SEG_PER_BLK):
                    su = sseg0 + u
                    v = plsc.load_gather(segmin, [su * L + lanes])
                    bm2 = jnp.minimum(bm2, jnp.where(su == sidx, sm2, v))
                plsc.store_scatter(blkmin, [bstar * L + lanes], bm2)

                pjx = plsc.load_gather(px_v, [jstar])
                pjy = plsc.load_gather(py_v, [jstar])
                pjz = plsc.load_gather(pz_v, [jstar])
                dx = wrap(rx - pjx)
                dy = wrap(ry - pjy)
                dz = wrap(rz - pjz)
                cx, cy, cz = cart_of(dx, dy, dz)

                d2c = jnp.maximum(m, zero)
                bits = plsc.bitcast(d2c, jnp.int32)
                y = plsc.bitcast(_splat_i(0x5F3759DF) - (bits >> 1), jnp.float32)
                hx = 0.5 * d2c
                y = y * (1.5 - hx * y * y)
                y = y * (1.5 - hx * y * y)
                sr = d2c * y
                sr = 0.5 * (sr + d2c / sr)
                dist = jnp.where(d2c > zero, sr, zero)

                ksp = jnp.full((L,), kk, jnp.int32)
                plsc.store_scatter(idx_st, [lanes, ksp], jstar)
                plsc.store_scatter(dist_st, [lanes, ksp], dist)
                k3 = ksp * 3
                plsc.store_scatter(vec_st, [lanes, k3], cx)
                plsc.store_scatter(vec_st, [lanes, k3 + 1], cy)
                plsc.store_scatter(vec_st, [lanes, k3 + 2], cz)
                return _
            lax.fori_loop(0, K, rank_body, None, unroll=False)

            pltpu.sync_copy(idx_st, oidx_h.at[pl.ds(row0, L)])
            pltpu.sync_copy(dist_st, odist_h.at[pl.ds(row0, L)])
            pltpu.sync_copy(vec_st, ovec_h.at[pl.ds(row0, L)])
            return _

        lax.fori_loop(0, NGROUPS, group_body, None, unroll=False)

    return knn_kernel(px, py, pz, cellv)


def kernel(positions, cell):
    px = positions[:, 0]
    py = positions[:, 1]
    pz = positions[:, 2]
    cellb = cell.astype(jnp.bfloat16).astype(jnp.float32)
    cellv = jnp.zeros((L,), jnp.float32).at[1:10].set(cellb.reshape(9))
    oidx, od, ov = _knn_sc(px, py, pz, cellv)
    src = jnp.broadcast_to(jnp.arange(N, dtype=jnp.int32)[:, None], (N, K))
    edge_index = jnp.stack([src.reshape(-1), oidx.reshape(-1)], axis=0)
    edge_vec = ov.reshape(N * K, 3)
    edge_dist = od.reshape(-1)
    return edge_index, edge_vec, edge_dist


# speedup vs baseline: 8.2485x; 1.0977x over previous
import functools

import jax
import jax.numpy as jnp
from jax import lax
from jax.experimental import pallas as pl
from jax.experimental.pallas import tpu as pltpu
from jax.experimental.pallas import tpu_sc as plsc

N = 2048
K = 19
NC = 2
NS = 16
L = 16
NW = NC * NS
ROWS_PER_W = N // NW
NGROUPS = ROWS_PER_W // L
NSEG = N // L
SEG_PER_BLK = 8
NBLK = NSEG // SEG_PER_BLK
BR = 256


def _splat_f(v):
    return jnp.full((L,), v, jnp.float32)


def _splat_i(v):
    return jnp.full((L,), v, jnp.int32)


def _d2_tc(pxr, pyr, pzr, pxc, pyc, pzc, cellb):
    def body(c_ref, pxr_ref, pyr_ref, pzr_ref, pxc_ref, pyc_ref, pzc_ref,
             o_ref):
        i = pl.program_id(0)

        def wrap(d):
            adj = jnp.where(jnp.abs(d) > 0.5, jnp.sign(d), 0.0)
            return d - adj

        def bf(d):
            bits = lax.bitcast_convert_type(d, jnp.int32)
            r = bits + (jnp.int32(0x7FFF) + ((bits >> 16) & 1))
            return lax.bitcast_convert_type(r & jnp.int32(-65536), jnp.float32)

        dx = bf(wrap(pxr_ref[...] - pxc_ref[...]))
        dy = bf(wrap(pyr_ref[...] - pyc_ref[...]))
        dz = bf(wrap(pzr_ref[...] - pzc_ref[...]))
        cx = dx * c_ref[0, 0] + dy * c_ref[1, 0] + dz * c_ref[2, 0]
        cy = dx * c_ref[0, 1] + dy * c_ref[1, 1] + dz * c_ref[2, 1]
        cz = dx * c_ref[0, 2] + dy * c_ref[1, 2] + dz * c_ref[2, 2]
        d2 = cx * cx + cy * cy + cz * cz
        rid = i * BR + lax.broadcasted_iota(jnp.int32, (BR, N), 0)
        cid = lax.broadcasted_iota(jnp.int32, (BR, N), 1)
        o_ref[...] = jnp.where(rid == cid, jnp.inf, d2)

    return pl.pallas_call(
        body,
        grid=(N // BR,),
        in_specs=[
            pl.BlockSpec(memory_space=pltpu.SMEM),
            pl.BlockSpec((BR, 1), lambda i: (i, 0)),
            pl.BlockSpec((BR, 1), lambda i: (i, 0)),
            pl.BlockSpec((BR, 1), lambda i: (i, 0)),
            pl.BlockSpec((1, N), lambda i: (0, 0)),
            pl.BlockSpec((1, N), lambda i: (0, 0)),
            pl.BlockSpec((1, N), lambda i: (0, 0)),
        ],
        out_specs=pl.BlockSpec((BR, N), lambda i: (i, 0)),
        out_shape=jax.ShapeDtypeStruct((N, N), jnp.float32),
    )(cellb, pxr, pyr, pzr, pxc, pyc, pzc)


def _knn_sc(d2m, px, py, pz, cellv):
    mesh = plsc.VectorSubcoreMesh(
        core_axis_name="c", subcore_axis_name="s", num_cores=NC, num_subcores=NS
    )

    @functools.partial(
        pl.kernel,
        out_type=(
            jax.ShapeDtypeStruct((N, K), jnp.int32),
            jax.ShapeDtypeStruct((N, K), jnp.float32),
            jax.ShapeDtypeStruct((N, 3 * K), jnp.float32),
        ),
        mesh=mesh,
        compiler_params=pltpu.CompilerParams(needs_layout_passes=False),
        scratch_types=[
            pltpu.VMEM((N,), jnp.float32),
            pltpu.VMEM((N,), jnp.float32),
            pltpu.VMEM((N,), jnp.float32),
            pltpu.VMEM((L,), jnp.float32),
            pltpu.VMEM((L, N), jnp.float32),
            pltpu.VMEM((NSEG * L,), jnp.float32),
            pltpu.VMEM((NBLK * L,), jnp.float32),
            pltpu.VMEM((L, K), jnp.int32),
            pltpu.VMEM((L, K), jnp.float32),
            pltpu.VMEM((L, 3 * K), jnp.float32),
        ],
    )
    def knn_kernel(d2_h, px_h, py_h, pz_h, cell_h, oidx_h, odist_h, ovec_h,
                   px_v, py_v, pz_v, cell_v, buf, segmin, blkmin,
                   idx_st, dist_st, vec_st):
        wid = lax.axis_index("s") * NC + lax.axis_index("c")
        pltpu.sync_copy(px_h, px_v)
        pltpu.sync_copy(py_h, py_v)
        pltpu.sync_copy(pz_h, pz_v)
        pltpu.sync_copy(cell_h, cell_v)

        lanes = lax.iota(jnp.int32, L)
        half = _splat_f(0.5)
        zero = _splat_f(0.0)
        inf = _splat_f(jnp.inf)

        def bget(e):
            return plsc.load_gather(cell_v, [_splat_i(e + 1)])

        c00, c01, c02 = bget(0), bget(1), bget(2)
        c10, c11, c12 = bget(3), bget(4), bget(5)
        c20, c21, c22 = bget(6), bget(7), bget(8)

        def wrap(d):
            adj = jnp.where(jnp.abs(d) > half, jnp.sign(d), zero)
            return d - adj

        def bf16r(x):
            bits = plsc.bitcast(x, jnp.int32)
            r = bits + (jnp.int32(0x7FFF) + ((bits >> 16) & 1))
            return plsc.bitcast(r & jnp.int32(-65536), jnp.float32)

        def cart_of(dx, dy, dz):
            dx, dy, dz = bf16r(dx), bf16r(dy), bf16r(dz)
            cx = dx * c00 + dy * c10 + dz * c20
            cy = dx * c01 + dy * c11 + dz * c21
            cz = dx * c02 + dy * c12 + dz * c22
            return cx, cy, cz

        def group_body(g, _):
            row0 = wid * ROWS_PER_W + g * L
            rx = px_v[pl.ds(row0, L)]
            ry = py_v[pl.ds(row0, L)]
            rz = pz_v[pl.ds(row0, L)]

            pltpu.sync_copy(d2_h.at[pl.ds(row0, L)], buf)

            def seg_body(s, _):
                sm = inf
                for t in range(L):
                    v = plsc.load_gather(buf, [lanes, _splat_i(0) + (s * L + t)])
                    sm = jnp.minimum(sm, v)
                segmin[pl.ds(s * L, L)] = sm
                return _
            lax.fori_loop(0, NSEG, seg_body, None, unroll=False)

            def blk_body(b, _):
                bm = inf
                for u in range(SEG_PER_BLK):
                    bm = jnp.minimum(bm, segmin[pl.ds((b * SEG_PER_BLK + u) * L, L)])
                blkmin[pl.ds(b * L, L)] = bm
                return _
            lax.fori_loop(0, NBLK, blk_body, None, unroll=False)

            def rank_body(kk, _):
                chains = []
                for c in range(4):
                    m = blkmin[pl.ds((4 * c) * L, L)]
                    bi = _splat_i(4 * c)
                    for b in range(4 * c + 1, 4 * c + 4):
                        v = blkmin[pl.ds(b * L, L)]
                        upd = v < m
                        m = jnp.where(upd, v, m)
                        bi = jnp.where(upd, _splat_i(b), bi)
                    chains.append((m, bi))
                m, bstar = chains[0]
                for cm, cb in chains[1:]:
                    upd = cm < m
                    m = jnp.where(upd, cm, m)
                    bstar = jnp.where(upd, cb, bstar)
                sseg0 = bstar * SEG_PER_BLK
                sv0 = plsc.load_gather(segmin, [sseg0 * L + lanes])
                sidx = sseg0
                for u in range(1, SEG_PER_BLK):
                    v = plsc.load_gather(segmin, [(sseg0 + u) * L + lanes])
                    upd = v < sv0
                    sv0 = jnp.where(upd, v, sv0)
                    sidx = jnp.where(upd, sseg0 + u, sidx)
                m = sv0

                sbase = sidx * L
                vts = [plsc.load_gather(buf, [lanes, sbase + t])
                       for t in range(L)]
                jstar = sbase + (L - 1)
                for t in range(L - 1, -1, -1):
                    jstar = jnp.where(vts[t] == m, sbase + t, jstar)

                plsc.store_scatter(buf, [lanes, jstar], inf)
                sm2 = inf
                for t in range(L):
                    sm2 = jnp.minimum(
                        sm2, jnp.where(sbase + t == jstar, inf, vts[t]))
                plsc.store_scatter(segmin, [sidx * L + lanes], sm2)
                bm2 = inf
                for u in range(SEG_PER_BLK):
                    su = sseg0 + u
                    v = plsc.load_gather(segmin, [su * L + lanes])
                    bm2 = jnp.minimum(bm2, jnp.where(su == sidx, sm2, v))
                plsc.store_scatter(blkmin, [bstar * L + lanes], bm2)

                pjx = plsc.load_gather(px_v, [jstar])
                pjy = plsc.load_gather(py_v, [jstar])
                pjz = plsc.load_gather(pz_v, [jstar])
                cx, cy, cz = cart_of(wrap(rx - pjx), wrap(ry - pjy),
                                     wrap(rz - pjz))

                d2c = jnp.maximum(m, zero)
                bits = plsc.bitcast(d2c, jnp.int32)
                y = plsc.bitcast(_splat_i(0x5F3759DF) - (bits >> 1), jnp.float32)
                hx = 0.5 * d2c
                y = y * (1.5 - hx * y * y)
                y = y * (1.5 - hx * y * y)
                sr = d2c * y
                sr = 0.5 * (sr + d2c / sr)
                dist = jnp.where(d2c > zero, sr, zero)

                ksp = jnp.full((L,), kk, jnp.int32)
                plsc.store_scatter(idx_st, [lanes, ksp], jstar)
                plsc.store_scatter(dist_st, [lanes, ksp], dist)
                k3 = ksp * 3
                plsc.store_scatter(vec_st, [lanes, k3], cx)
                plsc.store_scatter(vec_st, [lanes, k3 + 1], cy)
                plsc.store_scatter(vec_st, [lanes, k3 + 2], cz)
                return _
            lax.fori_loop(0, K, rank_body, None, unroll=False)

            pltpu.sync_copy(idx_st, oidx_h.at[pl.ds(row0, L)])
            pltpu.sync_copy(dist_st, odist_h.at[pl.ds(row0, L)])
            pltpu.sync_copy(vec_st, ovec_h.at[pl.ds(row0, L)])
            return _

        lax.fori_loop(0, NGROUPS, group_body, None, unroll=False)

    return knn_kernel(d2m, px, py, pz, cellv)


def kernel(positions, cell):
    px = positions[:, 0]
    py = positions[:, 1]
    pz = positions[:, 2]
    cb = lax.bitcast_convert_type(cell, jnp.int32)
    cb = (cb + (jnp.int32(0x7FFF) + ((cb >> 16) & 1))) & jnp.int32(-65536)
    cellb = lax.bitcast_convert_type(cb, jnp.float32)
    d2m = _d2_tc(px[:, None], py[:, None], pz[:, None],
                 px[None, :], py[None, :], pz[None, :], cellb)
    cellv = jnp.zeros((L,), jnp.float32).at[1:10].set(cellb.reshape(9))
    oidx, od, ov = _knn_sc(d2m, px, py, pz, cellv)
    src = jnp.broadcast_to(jnp.arange(N, dtype=jnp.int32)[:, None], (N, K))
    edge_index = jnp.stack([src.reshape(-1), oidx.reshape(-1)], axis=0)
    edge_vec = ov.reshape(N * K, 3)
    edge_dist = od.reshape(-1)
    return edge_index, edge_vec, edge_dist
